# baseline (speedup 1.0000x reference)
import functools

import jax
import jax.numpy as jnp
from jax import lax
from jax.experimental import pallas as pl
from jax.experimental.pallas import tpu as pltpu
from jax.experimental.pallas import tpu_sc as plsc

_NUM_EXPERTS = 64
_BLOCK_TOKENS = 4096
_LANES = 16


def _router_kernel(x_ref, w_ref, probs_ref):
    x = x_ref[0]
    logits = jax.lax.dot_general(
        w_ref[...], x,
        dimension_numbers=(((1,), (1,)), ((), ())),
        preferred_element_type=jnp.float32,
    )
    m1 = jnp.max(logits, axis=0, keepdims=True)
    e = jnp.exp(logits - m1)
    s = jnp.sum(e, axis=0, keepdims=True)
    probs_ref[0] = e / s


def _make_topk_sc(b, s):
    info = plsc.get_sparse_core_info()
    n_workers = info.num_cores * info.num_subcores
    tokens = b * s
    tok_per_w = tokens // n_workers
    chunk = 512
    n_chunks = tok_per_w // chunk
    slices_per_batch = s // tok_per_w
    mesh = plsc.VectorSubcoreMesh(core_axis_name="c", subcore_axis_name="s")

    @functools.partial(
        pl.kernel, mesh=mesh,
        out_type=[
            jax.ShapeDtypeStruct((b, 2, s), jnp.float32),
            jax.ShapeDtypeStruct((b, 2, s), jnp.int32),
        ],
        scratch_types=[
            pltpu.VMEM((_NUM_EXPERTS, chunk), jnp.float32),
            pltpu.VMEM((2, chunk), jnp.float32),
            pltpu.VMEM((2, chunk), jnp.int32),
        ],
    )
    def topk_sc(probs_hbm, w_hbm, i_hbm, p_v, w_v, i_v):
        wid = lax.axis_index("s") * info.num_cores + lax.axis_index("c")
        batch = wid // slices_per_batch
        base = (wid % slices_per_batch) * tok_per_w

        def do_chunk(ci, _):
            off = base + ci * chunk
            pltpu.sync_copy(probs_hbm.at[batch, :, pl.ds(off, chunk)], p_v)

            def do_group(g, _):
                col = pl.ds(g * _LANES, _LANES)
                m1 = p_v[0, col]
                i1 = jnp.zeros((_LANES,), jnp.int32)
                m2 = jnp.full((_LANES,), -1.0, jnp.float32)
                i2 = jnp.zeros((_LANES,), jnp.int32)
                for e in range(1, _NUM_EXPERTS):
                    v = p_v[e, col]
                    ev = jnp.full((_LANES,), e, jnp.int32)
                    gt1 = v > m1
                    gt2 = v > m2
                    m2n = jnp.where(gt2, v, m2)
                    i2n = jnp.where(gt2, ev, i2)
                    m2 = jnp.where(gt1, m1, m2n)
                    i2 = jnp.where(gt1, i1, i2n)
                    m1 = jnp.where(gt1, v, m1)
                    i1 = jnp.where(gt1, ev, i1)
                t = m1 + m2
                w_v[0, col] = m1 / t
                w_v[1, col] = m2 / t
                i_v[0, col] = i1
                i_v[1, col] = i2
                return 0

            lax.fori_loop(0, chunk // _LANES, do_group, 0, unroll=False)
            pltpu.sync_copy(w_v, w_hbm.at[batch, :, pl.ds(off, chunk)])
            pltpu.sync_copy(i_v, i_hbm.at[batch, :, pl.ds(off, chunk)])
            return 0

        lax.fori_loop(0, n_chunks, do_chunk, 0, unroll=False)

    return topk_sc


@jax.jit
def kernel(x, W):
    b, s, d = x.shape
    grid = (b, s // _BLOCK_TOKENS)
    probs_t = pl.pallas_call(
        _router_kernel,
        grid=grid,
        in_specs=[
            pl.BlockSpec((1, _BLOCK_TOKENS, d), lambda i, j: (i, j, 0)),
            pl.BlockSpec((_NUM_EXPERTS, d), lambda i, j: (0, 0)),
        ],
        out_specs=pl.BlockSpec((1, _NUM_EXPERTS, _BLOCK_TOKENS), lambda i, j: (i, 0, j)),
        out_shape=jax.ShapeDtypeStruct((b, _NUM_EXPERTS, s), jnp.float32),
        compiler_params=pltpu.CompilerParams(
            dimension_semantics=("parallel", "parallel"),
        ),
    )(x, W)
    weights_t, indices_t = _make_topk_sc(b, s)(probs_t)
    return (
        jnp.transpose(weights_t, (0, 2, 1)),
        jnp.transpose(indices_t, (0, 2, 1)),
        jnp.transpose(probs_t, (0, 2, 1)),
    )


# speedup vs baseline: 1.7785x; 1.7785x over previous
import jax
import jax.numpy as jnp
from jax.experimental import pallas as pl
from jax.experimental.pallas import tpu as pltpu

_NUM_EXPERTS = 64
_BLOCK_TOKENS = 4096


def _router_kernel(x_ref, w_ref, probs_ref, w_out_ref, i_out_ref):
    x = x_ref[0]
    logits = jax.lax.dot_general(
        w_ref[...], x,
        dimension_numbers=(((1,), (1,)), ((), ())),
        preferred_element_type=jnp.float32,
    )
    m1 = jnp.max(logits, axis=0, keepdims=True)
    e = jnp.exp(logits - m1)
    s = jnp.sum(e, axis=0, keepdims=True)
    probs_ref[0] = e / s

    iota = jax.lax.broadcasted_iota(jnp.int32, logits.shape, 0)
    i1 = jnp.argmax(logits, axis=0)
    masked = jnp.where(iota == i1[None, :], -jnp.inf, logits)
    m2 = jnp.max(masked, axis=0)
    i2 = jnp.argmax(masked, axis=0)

    r = jnp.exp(m2 - m1[0])
    w1 = 1.0 / (1.0 + r)
    w2 = r / (1.0 + r)
    w_out_ref[0] = jnp.stack([w1, w2], axis=0)
    i_out_ref[0] = jnp.stack([i1, i2], axis=0).astype(jnp.int32)


@jax.jit
def kernel(x, W):
    b, s, d = x.shape
    grid = (b, s // _BLOCK_TOKENS)
    probs_t, weights_t, indices_t = pl.pallas_call(
        _router_kernel,
        grid=grid,
        in_specs=[
            pl.BlockSpec((1, _BLOCK_TOKENS, d), lambda i, j: (i, j, 0)),
            pl.BlockSpec((_NUM_EXPERTS, d), lambda i, j: (0, 0)),
        ],
        out_specs=[
            pl.BlockSpec((1, _NUM_EXPERTS, _BLOCK_TOKENS), lambda i, j: (i, 0, j)),
            pl.BlockSpec((1, 2, _BLOCK_TOKENS), lambda i, j: (i, 0, j)),
            pl.BlockSpec((1, 2, _BLOCK_TOKENS), lambda i, j: (i, 0, j)),
        ],
        out_shape=[
            jax.ShapeDtypeStruct((b, _NUM_EXPERTS, s), jnp.float32),
            jax.ShapeDtypeStruct((b, 2, s), jnp.float32),
            jax.ShapeDtypeStruct((b, 2, s), jnp.int32),
        ],
        compiler_params=pltpu.CompilerParams(
            dimension_semantics=("parallel", "parallel"),
        ),
    )(x, W)
    return (
        jnp.transpose(weights_t, (0, 2, 1)),
        jnp.transpose(indices_t, (0, 2, 1)),
        jnp.transpose(probs_t, (0, 2, 1)),
    )
